# baseline (device time: 16289 ns/iter reference)
import jax
import jax.numpy as jnp
from jax import lax
from jax.experimental import pallas as pl
from jax.experimental.pallas import tpu as pltpu

N_Z = 4
BLOCK_M = 512


def kernel(x, dy, gamma):
    del gamma
    m, d = x.shape
    n_blocks = m // BLOCK_M

    h = d // 2

    def body(xl_ref, xr_ref, dyl_ref, dyr_ref, out_ref, comm_ref,
             send_sems, recv_sems):
        step = pl.program_id(0)
        my_x = lax.axis_index("x")
        my_y = lax.axis_index("y")
        my_z = lax.axis_index("z")

        xl = xl_ref[...]
        xr = xr_ref[...]
        dyl = dyl_ref[...]
        dyr = dyr_ref[...]
        mu = (jnp.sum(xl, axis=1, keepdims=True)
              + jnp.sum(xr, axis=1, keepdims=True)) / d
        xcl = xl - mu
        xcr = xr - mu
        var = (jnp.sum(xcl * xcl, axis=1, keepdims=True)
               + jnp.sum(xcr * xcr, axis=1, keepdims=True)) / d
        rstd = lax.rsqrt(var + 1e-5)
        ones = jnp.ones((1, BLOCK_M), jnp.float32)
        for cols, xc, dyb in ((slice(0, h), xcl, dyl),
                              (slice(h, d), xcr, dyr)):
            t = dyb * (xc * rstd)
            dg = jnp.dot(ones, t, preferred_element_type=jnp.float32)
            db = jnp.dot(ones, dyb, preferred_element_type=jnp.float32)
            part = jnp.concatenate([dg, db], axis=0)

            @pl.when(step == 0)
            def _():
                comm_ref[0, :, cols] = part

            @pl.when(step > 0)
            def _():
                comm_ref[0, :, cols] = comm_ref[0, :, cols] + part

        @pl.when(step == n_blocks - 1)
        def _():
            out_ref[...] = comm_ref[0]

    return pl.pallas_call(
        body,
        grid=(n_blocks,),
        out_shape=jax.ShapeDtypeStruct((2, d), jnp.float32),
        in_specs=[
            pl.BlockSpec((BLOCK_M, h), lambda i: (i, 0)),
            pl.BlockSpec((BLOCK_M, h), lambda i: (i, 1)),
            pl.BlockSpec((BLOCK_M, h), lambda i: (i, 0)),
            pl.BlockSpec((BLOCK_M, h), lambda i: (i, 1)),
        ],
        out_specs=pl.BlockSpec((2, d), lambda i: (0, 0)),
        scratch_shapes=[
            pltpu.VMEM((N_Z, 2, d), jnp.float32),
            pltpu.SemaphoreType.DMA((N_Z - 1,)),
            pltpu.SemaphoreType.DMA((N_Z - 1,)),
        ],
        compiler_params=pltpu.CompilerParams(
            dimension_semantics=("arbitrary",),
        ),
    )(x, x, dy, dy)
